# Initial kernel scaffold; baseline (speedup 1.0000x reference)
import jax
import jax.numpy as jnp
from jax.experimental import pallas as pl


def kernel(x, edge_index, W_lin, W_att, b_att):
    raise NotImplementedError("write your pallas kernel here")


# speedup vs baseline: 3.4717x; 3.4717x over previous
import functools

import jax
import jax.numpy as jnp
from jax import lax
from jax.experimental import pallas as pl
from jax.experimental.pallas import tpu as pltpu
from jax.experimental.pallas import tpu_sc as plsc

N = 10000
E = 320000
D = 128
NCORES = 2
NSUB = 16
HALF = N // NCORES
ACC_ROWS = 5120
IPT = ACC_ROWS // NSUB
WPT = 312
KB = 128
NBLK = E // KB


def _tc_precompute(x, wbig, bias):
    def body(x_ref, w_ref, b_ref, o_ref):
        o_ref[...] = (
            jnp.dot(x_ref[...], w_ref[...], preferred_element_type=jnp.float32)
            + b_ref[...]
        )

    return pl.pallas_call(
        body,
        grid=(5,),
        in_specs=[
            pl.BlockSpec((N // 5, D), lambda i: (i, 0)),
            pl.BlockSpec((D, 2 * D), lambda i: (0, 0)),
            pl.BlockSpec((1, 2 * D), lambda i: (0, 0)),
        ],
        out_specs=pl.BlockSpec((N // 5, 2 * D), lambda i: (i, 0)),
        out_shape=jax.ShapeDtypeStruct((N, 2 * D), jnp.float32),
    )(x, wbig, bias)


@functools.partial(
    pl.kernel,
    out_type=[
        jax.ShapeDtypeStruct((N, D), jnp.float32),
        jax.ShapeDtypeStruct((N, D), jnp.float32),
    ],
    mesh=plsc.VectorSubcoreMesh(core_axis_name="c", subcore_axis_name="s"),
    compiler_params=pltpu.CompilerParams(needs_layout_passes=False),
    scratch_types=[
        pltpu.VMEM((KB,), jnp.int32),
        pltpu.VMEM((KB,), jnp.int32),
        pltpu.VMEM((KB,), jnp.int32),
        pltpu.VMEM((KB,), jnp.float32),
        pltpu.VMEM((KB,), jnp.float32),
        pltpu.VMEM((KB,), jnp.float32),
        pltpu.VMEM((KB, D), jnp.float32),
        pltpu.VMEM((KB, D), jnp.float32),
        pltpu.VMEM_SHARED((ACC_ROWS, D), jnp.float32),
        pltpu.VMEM_SHARED((ACC_ROWS, D), jnp.float32),
    ],
)
def _sc_edges(src_hbm, dest_hbm, s1_hbm, s2_hbm, y_hbm,
              feat_out, att_out,
              src_v, dest_v, adj_v, s1_v, s2_v, scores_v, rows_v, abs_v,
              feat_acc, att_acc):
    core = lax.axis_index("c")
    sub = lax.axis_index("s")
    base = core * HALF

    def zbody(r, _):
        z = jnp.zeros((16,), jnp.float32)
        for j in range(D // 16):
            rows_v[r, pl.ds(j * 16, 16)] = z
        return 0
    lax.fori_loop(0, KB, zbody, 0)
    r0 = sub * IPT
    for acc in (feat_acc, att_acc):
        pltpu.sync_copy(rows_v, acc.at[pl.ds(r0, KB)])
        pltpu.sync_copy(rows_v, acc.at[pl.ds(r0 + KB, KB)])
        pltpu.sync_copy(rows_v.at[pl.ds(0, IPT - 2 * KB)],
                        acc.at[pl.ds(r0 + 2 * KB, IPT - 2 * KB)])

    plsc.subcore_barrier()

    nblk_t = (NBLK - 1 - sub) // NSUB + 1

    def block_body(i, _):
        blk = sub + i * NSUB
        off = blk * KB
        pltpu.sync_copy(src_hbm.at[pl.ds(off, KB)], src_v)
        pltpu.sync_copy(dest_hbm.at[pl.ds(off, KB)], dest_v)
        pltpu.sync_copy(y_hbm.at[src_v], rows_v)
        pltpu.sync_copy(s1_hbm.at[src_v], s1_v)
        pltpu.sync_copy(s2_hbm.at[dest_v], s2_v)

        for j in range(KB // 16):
            sl = pl.ds(j * 16, 16)
            sc = s1_v[sl] + s2_v[sl]
            sc = jnp.where(sc >= 0.0, sc, sc * 0.01)
            scores_v[sl] = sc
            adj = dest_v[sl] - base
            ok = (adj >= 0) & (adj < HALF)
            adj_v[sl] = jnp.where(ok, adj, HALF)

        def gbody(g, _):
            scv = scores_v[pl.ds(g * 16, 16)]
            for e16 in range(16):
                e = g * 16 + e16
                s = scv[e16]
                for j in range(D // 16):
                    sl = pl.ds(j * 16, 16)
                    w = rows_v[e, sl] * s
                    rows_v[e, sl] = w
                    abs_v[e, sl] = jnp.abs(w)
            return 0
        lax.fori_loop(0, KB // 16, gbody, 0)

        pltpu.sync_copy(rows_v, feat_acc.at[adj_v], add=True)
        pltpu.sync_copy(abs_v, att_acc.at[adj_v], add=True)
        return 0

    lax.fori_loop(0, nblk_t, block_body, 0)

    plsc.subcore_barrier()

    @pl.when(sub < NSUB - 1)
    def _():
        w0 = sub * WPT
        pltpu.sync_copy(feat_acc.at[pl.ds(w0, WPT)],
                        feat_out.at[pl.ds(base + w0, WPT)])
        pltpu.sync_copy(att_acc.at[pl.ds(w0, WPT)],
                        att_out.at[pl.ds(base + w0, WPT)])

    @pl.when(sub == NSUB - 1)
    def _():
        w0 = (NSUB - 1) * WPT
        pltpu.sync_copy(feat_acc.at[pl.ds(w0, HALF - w0)],
                        feat_out.at[pl.ds(base + w0, HALF - w0)])
        pltpu.sync_copy(att_acc.at[pl.ds(w0, HALF - w0)],
                        att_out.at[pl.ds(base + w0, HALF - w0)])


def kernel(x, edge_index, W_lin, W_att, b_att):
    a1 = W_att[0, :D]
    a2 = W_att[0, D:]
    wbig = jnp.concatenate(
        [W_lin.T, a1[:, None], a2[:, None],
         jnp.zeros((D, 2 * D - D - 2), jnp.float32)], axis=1)
    bias = jnp.zeros((1, 2 * D), jnp.float32).at[0, D].set(b_att[0])
    Y = _tc_precompute(x, wbig, bias)
    y = Y[:, :D]
    s1 = Y[:, D]
    s2 = Y[:, D + 1]
    src = edge_index[0]
    dest = edge_index[1]
    feat, att = _sc_edges(src, dest, s1, s2, y)
    return feat, att
